# baseline (speedup 1.0000x reference)
import jax
import jax.numpy as jnp
from jax import lax
from jax.experimental import pallas as pl
from jax.experimental.pallas import tpu as pltpu
from jax.experimental.pallas import tpu_sc as plsc

N_NODE = 16384
N_EDGE = 4096
D = 32
LANES = 128

_NC = 2
_NS = 16
_NW = _NC * _NS

_NODE_PER_W = N_NODE // _NW
_EDGE_PER_W = N_EDGE // _NW


def _slab_gather(idx_v, tabT, rowbuf_v, n, slabs_v, gsem):
    rows = lax.iota(jnp.int32, 16)

    def _fire(vec, l, bank):
        idx = vec[l]
        lane0 = pl.multiple_of((idx >> 7) << 7, LANES)
        pltpu.async_copy(tabT.at[:, pl.ds(lane0, LANES)],
                         slabs_v.at[pl.ds((bank * 4 + l % 4) * D, D)], gsem)

    def _wait_bank(bank):
        for l in range(4):
            pltpu.make_async_copy(
                tabT.at[:, pl.ds(0, LANES)],
                slabs_v.at[pl.ds((bank * 4 + l) * D, D)], gsem).wait()

    def _extract(vec, l, bank, slot):
        base_r = (bank * 4 + l % 4) * D
        c = (rows & 0) + (vec[l] & (LANES - 1))
        lo = plsc.load_gather(slabs_v, [rows + base_r, c])
        hi = plsc.load_gather(slabs_v, [rows + (base_r + 16), c])
        row = rowbuf_v.at[slot]
        row[pl.ds(0, 16)] = lo
        row[pl.ds(16, 16)] = hi

    def _chunk(g, _):
        vec = idx_v[pl.ds(g * 16, 16)]
        for l in range(4):
            _fire(vec, l, 0)
        for l in range(4, 8):
            _fire(vec, l, 1)
        _wait_bank(0)
        for l in range(4):
            _extract(vec, l, 0, g * 16 + l)
        for l in range(8, 12):
            _fire(vec, l, 0)
        _wait_bank(1)
        for l in range(4, 8):
            _extract(vec, l, 1, g * 16 + l)
        for l in range(12, 16):
            _fire(vec, l, 1)
        _wait_bank(0)
        for l in range(8, 12):
            _extract(vec, l, 0, g * 16 + l)
        _wait_bank(1)
        for l in range(12, 16):
            _extract(vec, l, 1, g * 16 + l)
        return _

    lax.fori_loop(0, n // 16, _chunk, 0)


N_SPLIT = 4
N_Q = N_NODE // N_SPLIT
_Q_PER_W = N_Q // _NW


def _gather_a_body(node_idx, edge_idx, node_tabT, edge_tabT,
                   node_out, edge_out,
                   nidx_v, eidx_v, nrow_v, erow_v, slabs_v, gsem):
    wid = lax.axis_index("s") * _NC + lax.axis_index("c")
    nbase = wid * _Q_PER_W
    ebase = wid * _EDGE_PER_W

    pltpu.sync_copy(node_idx.at[pl.ds(nbase, _Q_PER_W)], nidx_v)
    pltpu.sync_copy(edge_idx.at[pl.ds(ebase, _EDGE_PER_W)], eidx_v)

    _slab_gather(eidx_v, edge_tabT, erow_v, _EDGE_PER_W, slabs_v, gsem)
    pltpu.sync_copy(erow_v, edge_out.at[pl.ds(ebase, _EDGE_PER_W)])

    _slab_gather(nidx_v, node_tabT, nrow_v, _Q_PER_W, slabs_v, gsem)
    pltpu.sync_copy(nrow_v, node_out.at[pl.ds(nbase, _Q_PER_W)])


def _gather_b_body(node_idx, node_tabT, node_out,
                   nidx_v, nrow_v, slabs_v, gsem):
    wid = lax.axis_index("s") * _NC + lax.axis_index("c")
    nbase = wid * _Q_PER_W

    pltpu.sync_copy(node_idx.at[pl.ds(nbase, _Q_PER_W)], nidx_v)
    _slab_gather(nidx_v, node_tabT, nrow_v, _Q_PER_W, slabs_v, gsem)
    pltpu.sync_copy(nrow_v, node_out.at[pl.ds(nbase, _Q_PER_W)])


_gather_a = pl.kernel(
    _gather_a_body,
    out_type=(
        jax.ShapeDtypeStruct((N_Q, D), jnp.float32),
        jax.ShapeDtypeStruct((N_EDGE, D), jnp.float32),
    ),
    mesh=plsc.VectorSubcoreMesh(core_axis_name="c", subcore_axis_name="s"),
    compiler_params=pltpu.CompilerParams(needs_layout_passes=False),
    scratch_types=[
        pltpu.VMEM((_Q_PER_W,), jnp.int32),
        pltpu.VMEM((_EDGE_PER_W,), jnp.int32),
        pltpu.VMEM((_Q_PER_W, D), jnp.float32),
        pltpu.VMEM((_EDGE_PER_W, D), jnp.float32),
        pltpu.VMEM((8 * D, LANES), jnp.float32),
        pltpu.SemaphoreType.DMA,
    ],
)

_gather_b = pl.kernel(
    _gather_b_body,
    out_type=jax.ShapeDtypeStruct((N_Q, D), jnp.float32),
    mesh=plsc.VectorSubcoreMesh(core_axis_name="c", subcore_axis_name="s"),
    compiler_params=pltpu.CompilerParams(needs_layout_passes=False),
    scratch_types=[
        pltpu.VMEM((_Q_PER_W,), jnp.int32),
        pltpu.VMEM((_Q_PER_W, D), jnp.float32),
        pltpu.VMEM((8 * D, LANES), jnp.float32),
        pltpu.SemaphoreType.DMA,
    ],
)


M_BLK = 512


def _mm_body(node_ref, edge_ref, out_ref, j_ref):
    i = pl.program_id(0)
    out_ref[...] = lax.dot_general(
        node_ref[...], edge_ref[...],
        (((1,), (1,)), ((), ())),
        preferred_element_type=jnp.float32,
    )

    @pl.when(i == 0)
    def _():
        j_ref[...] = jnp.sum(edge_ref[...], axis=0, keepdims=True) * (1.0 / N_EDGE)


_matmul_a = pl.pallas_call(
    _mm_body,
    grid=(N_Q // M_BLK,),
    in_specs=[
        pl.BlockSpec((M_BLK, D), lambda i: (i, 0)),
        pl.BlockSpec((N_EDGE, D), lambda i: (0, 0)),
    ],
    out_specs=[
        pl.BlockSpec((M_BLK, N_EDGE), lambda i: (i, 0)),
        pl.BlockSpec((1, D), lambda i: (0, 0)),
    ],
    out_shape=[
        jax.ShapeDtypeStruct((N_NODE, N_EDGE), jnp.float32),
        jax.ShapeDtypeStruct((1, D), jnp.float32),
    ],
)


def _mm_b_body(node_ref, edge_ref, prev_ref, out_ref):
    out_ref[...] = lax.dot_general(
        node_ref[...], edge_ref[...],
        (((1,), (1,)), ((), ())),
        preferred_element_type=jnp.float32,
    )


def _make_matmul_k(k):
    return pl.pallas_call(
        _mm_b_body,
        grid=(N_Q // M_BLK,),
        in_specs=[
            pl.BlockSpec((M_BLK, D), lambda i: (i, 0)),
            pl.BlockSpec((N_EDGE, D), lambda i: (0, 0)),
            pl.BlockSpec(memory_space=pltpu.MemorySpace.HBM),
        ],
        out_specs=pl.BlockSpec((M_BLK, N_EDGE),
                               lambda i, k=k: (i + k * (N_Q // M_BLK), 0)),
        out_shape=jax.ShapeDtypeStruct((N_NODE, N_EDGE), jnp.float32),
        input_output_aliases={2: 0},
    )


_matmul_ks = [_make_matmul_k(k) for k in range(1, N_SPLIT)]


def kernel(node_labels, hyperedge_labels, embedding, edge_embedding):
    node_tabT = embedding.T
    edge_tabT = edge_embedding.T
    nodes_0, edge_embeds = _gather_a(node_labels[:N_Q], hyperedge_labels,
                                     node_tabT, edge_tabT)
    nodes_k = [_gather_b(node_labels[k * N_Q:(k + 1) * N_Q], node_tabT)
               for k in range(1, N_SPLIT)]
    out, j2d = _matmul_a(nodes_0, edge_embeds)
    for k in range(1, N_SPLIT):
        out = _matmul_ks[k - 1](nodes_k[k - 1], edge_embeds, out)
    return out, j2d.reshape(D)


# speedup vs baseline: 1.0158x; 1.0158x over previous
import jax
import jax.numpy as jnp
from jax import lax
from jax.experimental import pallas as pl
from jax.experimental.pallas import tpu as pltpu
from jax.experimental.pallas import tpu_sc as plsc

N_NODE = 16384
N_EDGE = 4096
D = 32
LANES = 128

_NC = 2
_NS = 16
_NW = _NC * _NS

_NODE_PER_W = N_NODE // _NW
_EDGE_PER_W = N_EDGE // _NW


def _slab_gather(idx_v, tabT, rowbuf_v, n, slabs_v, gsem):
    rows = lax.iota(jnp.int32, 16)

    def _fire(vec, l, bank):
        idx = vec[l]
        lane0 = pl.multiple_of((idx >> 7) << 7, LANES)
        pltpu.async_copy(tabT.at[:, pl.ds(lane0, LANES)],
                         slabs_v.at[pl.ds((bank * 4 + l % 4) * D, D)], gsem)

    def _wait_bank(bank):
        for l in range(4):
            pltpu.make_async_copy(
                tabT.at[:, pl.ds(0, LANES)],
                slabs_v.at[pl.ds((bank * 4 + l) * D, D)], gsem).wait()

    def _extract(vec, l, bank, slot):
        base_r = (bank * 4 + l % 4) * D
        c = (rows & 0) + (vec[l] & (LANES - 1))
        lo = plsc.load_gather(slabs_v, [rows + base_r, c])
        hi = plsc.load_gather(slabs_v, [rows + (base_r + 16), c])
        row = rowbuf_v.at[slot]
        row[pl.ds(0, 16)] = lo
        row[pl.ds(16, 16)] = hi

    def _chunk(g, _):
        vec = idx_v[pl.ds(g * 16, 16)]
        for l in range(4):
            _fire(vec, l, 0)
        for l in range(4, 8):
            _fire(vec, l, 1)
        _wait_bank(0)
        for l in range(4):
            _extract(vec, l, 0, g * 16 + l)
        for l in range(8, 12):
            _fire(vec, l, 0)
        _wait_bank(1)
        for l in range(4, 8):
            _extract(vec, l, 1, g * 16 + l)
        for l in range(12, 16):
            _fire(vec, l, 1)
        _wait_bank(0)
        for l in range(8, 12):
            _extract(vec, l, 0, g * 16 + l)
        _wait_bank(1)
        for l in range(12, 16):
            _extract(vec, l, 1, g * 16 + l)
        return _

    lax.fori_loop(0, n // 16, _chunk, 0)


N_A = 12288
N_B = N_NODE - N_A
_A_PER_W = N_A // _NW
_B_PER_W = N_B // _NW


def _gather_a_body(node_idx, edge_idx, node_tabT, edge_tabT,
                   node_out, edge_out,
                   nidx_v, eidx_v, nrow_v, erow_v, slabs_v, gsem):
    wid = lax.axis_index("s") * _NC + lax.axis_index("c")
    nbase = wid * _A_PER_W
    ebase = wid * _EDGE_PER_W

    pltpu.sync_copy(node_idx.at[pl.ds(nbase, _A_PER_W)], nidx_v)
    pltpu.sync_copy(edge_idx.at[pl.ds(ebase, _EDGE_PER_W)], eidx_v)

    _slab_gather(eidx_v, edge_tabT, erow_v, _EDGE_PER_W, slabs_v, gsem)
    pltpu.sync_copy(erow_v, edge_out.at[pl.ds(ebase, _EDGE_PER_W)])

    _slab_gather(nidx_v, node_tabT, nrow_v, _A_PER_W, slabs_v, gsem)
    pltpu.sync_copy(nrow_v, node_out.at[pl.ds(nbase, _A_PER_W)])


def _gather_b_body(node_idx, node_tabT, node_out,
                   nidx_v, nrow_v, slabs_v, gsem):
    wid = lax.axis_index("s") * _NC + lax.axis_index("c")
    nbase = wid * _B_PER_W

    pltpu.sync_copy(node_idx.at[pl.ds(nbase, _B_PER_W)], nidx_v)
    _slab_gather(nidx_v, node_tabT, nrow_v, _B_PER_W, slabs_v, gsem)
    pltpu.sync_copy(nrow_v, node_out.at[pl.ds(nbase, _B_PER_W)])


_gather_a = pl.kernel(
    _gather_a_body,
    out_type=(
        jax.ShapeDtypeStruct((N_A, D), jnp.float32),
        jax.ShapeDtypeStruct((N_EDGE, D), jnp.float32),
    ),
    mesh=plsc.VectorSubcoreMesh(core_axis_name="c", subcore_axis_name="s"),
    compiler_params=pltpu.CompilerParams(needs_layout_passes=False),
    scratch_types=[
        pltpu.VMEM((_A_PER_W,), jnp.int32),
        pltpu.VMEM((_EDGE_PER_W,), jnp.int32),
        pltpu.VMEM((_A_PER_W, D), jnp.float32),
        pltpu.VMEM((_EDGE_PER_W, D), jnp.float32),
        pltpu.VMEM((8 * D, LANES), jnp.float32),
        pltpu.SemaphoreType.DMA,
    ],
)

_gather_b = pl.kernel(
    _gather_b_body,
    out_type=jax.ShapeDtypeStruct((N_B, D), jnp.float32),
    mesh=plsc.VectorSubcoreMesh(core_axis_name="c", subcore_axis_name="s"),
    compiler_params=pltpu.CompilerParams(needs_layout_passes=False),
    scratch_types=[
        pltpu.VMEM((_B_PER_W,), jnp.int32),
        pltpu.VMEM((_B_PER_W, D), jnp.float32),
        pltpu.VMEM((8 * D, LANES), jnp.float32),
        pltpu.SemaphoreType.DMA,
    ],
)


M_BLK = 512


def _mm_body(node_ref, edge_ref, out_ref, j_ref):
    i = pl.program_id(0)
    out_ref[...] = lax.dot_general(
        node_ref[...], edge_ref[...],
        (((1,), (1,)), ((), ())),
        preferred_element_type=jnp.float32,
    )

    @pl.when(i == 0)
    def _():
        j_ref[...] = jnp.sum(edge_ref[...], axis=0, keepdims=True) * (1.0 / N_EDGE)


_matmul_a = pl.pallas_call(
    _mm_body,
    grid=(N_A // M_BLK,),
    in_specs=[
        pl.BlockSpec((M_BLK, D), lambda i: (i, 0)),
        pl.BlockSpec((N_EDGE, D), lambda i: (0, 0)),
    ],
    out_specs=[
        pl.BlockSpec((M_BLK, N_EDGE), lambda i: (i, 0)),
        pl.BlockSpec((1, D), lambda i: (0, 0)),
    ],
    out_shape=[
        jax.ShapeDtypeStruct((N_NODE, N_EDGE), jnp.float32),
        jax.ShapeDtypeStruct((1, D), jnp.float32),
    ],
)


def _mm_b_body(node_ref, edge_ref, prev_ref, out_ref):
    out_ref[...] = lax.dot_general(
        node_ref[...], edge_ref[...],
        (((1,), (1,)), ((), ())),
        preferred_element_type=jnp.float32,
    )


_matmul_b = pl.pallas_call(
    _mm_b_body,
    grid=(N_B // M_BLK,),
    in_specs=[
        pl.BlockSpec((M_BLK, D), lambda i: (i, 0)),
        pl.BlockSpec((N_EDGE, D), lambda i: (0, 0)),
        pl.BlockSpec(memory_space=pltpu.MemorySpace.HBM),
    ],
    out_specs=pl.BlockSpec((M_BLK, N_EDGE),
                           lambda i: (i + N_A // M_BLK, 0)),
    out_shape=jax.ShapeDtypeStruct((N_NODE, N_EDGE), jnp.float32),
    input_output_aliases={2: 0},
)


def kernel(node_labels, hyperedge_labels, embedding, edge_embedding):
    node_tabT = embedding.T
    edge_tabT = edge_embedding.T
    nodes_a, edge_embeds = _gather_a(node_labels[:N_A], hyperedge_labels,
                                     node_tabT, edge_tabT)
    nodes_b = _gather_b(node_labels[N_A:], node_tabT)
    out1, j2d = _matmul_a(nodes_a, edge_embeds)
    recon_logits = _matmul_b(nodes_b, edge_embeds, out1)
    return recon_logits, j2d.reshape(D)


# speedup vs baseline: 1.0234x; 1.0074x over previous
import jax
import jax.numpy as jnp
from jax import lax
from jax.experimental import pallas as pl
from jax.experimental.pallas import tpu as pltpu
from jax.experimental.pallas import tpu_sc as plsc

N_NODE = 16384
N_EDGE = 4096
D = 32
LANES = 128

_NC = 2
_NS = 16
_NW = _NC * _NS

_NODE_PER_W = N_NODE // _NW
_EDGE_PER_W = N_EDGE // _NW


def _slab_gather(idx_v, tabT, rowbuf_v, n, slabs_v, gsem):
    rows = lax.iota(jnp.int32, 16)

    def _fire(vec, l, bank):
        idx = vec[l]
        lane0 = pl.multiple_of((idx >> 7) << 7, LANES)
        pltpu.async_copy(tabT.at[:, pl.ds(lane0, LANES)],
                         slabs_v.at[pl.ds((bank * 4 + l % 4) * D, D)], gsem)

    def _wait_bank(bank):
        for l in range(4):
            pltpu.make_async_copy(
                tabT.at[:, pl.ds(0, LANES)],
                slabs_v.at[pl.ds((bank * 4 + l) * D, D)], gsem).wait()

    def _extract(vec, l, bank, slot):
        base_r = (bank * 4 + l % 4) * D
        c = (rows & 0) + (vec[l] & (LANES - 1))
        lo = plsc.load_gather(slabs_v, [rows + base_r, c])
        hi = plsc.load_gather(slabs_v, [rows + (base_r + 16), c])
        row = rowbuf_v.at[slot]
        row[pl.ds(0, 16)] = lo
        row[pl.ds(16, 16)] = hi

    def _chunk(g, _):
        vec = idx_v[pl.ds(g * 16, 16)]
        for l in range(4):
            _fire(vec, l, 0)
        for l in range(4, 8):
            _fire(vec, l, 1)
        _wait_bank(0)
        for l in range(4):
            _extract(vec, l, 0, g * 16 + l)
        for l in range(8, 12):
            _fire(vec, l, 0)
        _wait_bank(1)
        for l in range(4, 8):
            _extract(vec, l, 1, g * 16 + l)
        for l in range(12, 16):
            _fire(vec, l, 1)
        _wait_bank(0)
        for l in range(8, 12):
            _extract(vec, l, 0, g * 16 + l)
        _wait_bank(1)
        for l in range(12, 16):
            _extract(vec, l, 1, g * 16 + l)
        return _

    lax.fori_loop(0, n // 16, _chunk, 0)


N_HALF = N_NODE // 2
_HALF_PER_W = N_HALF // _NW


def _gather_a_body(node_idx, edge_idx, node_tabT, edge_tabT,
                   node_out, edge_out,
                   nidx_v, eidx_v, nrow_v, erow_v, slabs_v, gsem):
    wid = lax.axis_index("s") * _NC + lax.axis_index("c")
    nbase = wid * _HALF_PER_W
    ebase = wid * _EDGE_PER_W

    pltpu.sync_copy(node_idx.at[pl.ds(nbase, _HALF_PER_W)], nidx_v)
    pltpu.sync_copy(edge_idx.at[pl.ds(ebase, _EDGE_PER_W)], eidx_v)

    _slab_gather(eidx_v, edge_tabT, erow_v, _EDGE_PER_W, slabs_v, gsem)
    pltpu.sync_copy(erow_v, edge_out.at[pl.ds(ebase, _EDGE_PER_W)])

    _slab_gather(nidx_v, node_tabT, nrow_v, _HALF_PER_W, slabs_v, gsem)
    pltpu.sync_copy(nrow_v, node_out.at[pl.ds(nbase, _HALF_PER_W)])


def _gather_b_body(node_idx, node_tabT, node_out,
                   nidx_v, nrow_v, slabs_v, gsem):
    wid = lax.axis_index("s") * _NC + lax.axis_index("c")
    nbase = wid * _HALF_PER_W

    pltpu.sync_copy(node_idx.at[pl.ds(nbase, _HALF_PER_W)], nidx_v)
    _slab_gather(nidx_v, node_tabT, nrow_v, _HALF_PER_W, slabs_v, gsem)
    pltpu.sync_copy(nrow_v, node_out.at[pl.ds(nbase, _HALF_PER_W)])


_gather_a = pl.kernel(
    _gather_a_body,
    out_type=(
        jax.ShapeDtypeStruct((N_HALF, D), jnp.float32),
        jax.ShapeDtypeStruct((N_EDGE, D), jnp.float32),
    ),
    mesh=plsc.VectorSubcoreMesh(core_axis_name="c", subcore_axis_name="s"),
    compiler_params=pltpu.CompilerParams(needs_layout_passes=False),
    scratch_types=[
        pltpu.VMEM((_HALF_PER_W,), jnp.int32),
        pltpu.VMEM((_EDGE_PER_W,), jnp.int32),
        pltpu.VMEM((_HALF_PER_W, D), jnp.float32),
        pltpu.VMEM((_EDGE_PER_W, D), jnp.float32),
        pltpu.VMEM((8 * D, LANES), jnp.float32),
        pltpu.SemaphoreType.DMA,
    ],
)

_gather_b = pl.kernel(
    _gather_b_body,
    out_type=jax.ShapeDtypeStruct((N_HALF, D), jnp.float32),
    mesh=plsc.VectorSubcoreMesh(core_axis_name="c", subcore_axis_name="s"),
    compiler_params=pltpu.CompilerParams(needs_layout_passes=False),
    scratch_types=[
        pltpu.VMEM((_HALF_PER_W,), jnp.int32),
        pltpu.VMEM((_HALF_PER_W, D), jnp.float32),
        pltpu.VMEM((8 * D, LANES), jnp.float32),
        pltpu.SemaphoreType.DMA,
    ],
)


M_BLK = 512


def _mm_body(node_ref, edge_ref, out_ref, j_ref):
    i = pl.program_id(0)
    out_ref[...] = lax.dot_general(
        node_ref[...], edge_ref[...],
        (((1,), (1,)), ((), ())),
        preferred_element_type=jnp.float32,
    )

    @pl.when(i == 0)
    def _():
        j_ref[...] = jnp.sum(edge_ref[...], axis=0, keepdims=True) * (1.0 / N_EDGE)


_matmul_a = pl.pallas_call(
    _mm_body,
    grid=(N_HALF // M_BLK,),
    in_specs=[
        pl.BlockSpec((M_BLK, D), lambda i: (i, 0)),
        pl.BlockSpec((N_EDGE, D), lambda i: (0, 0)),
    ],
    out_specs=[
        pl.BlockSpec((M_BLK, N_EDGE), lambda i: (i, 0)),
        pl.BlockSpec((1, D), lambda i: (0, 0)),
    ],
    out_shape=[
        jax.ShapeDtypeStruct((N_NODE, N_EDGE), jnp.float32),
        jax.ShapeDtypeStruct((1, D), jnp.float32),
    ],
)


def _mm_b_body(node_ref, edge_ref, prev_ref, out_ref):
    out_ref[...] = lax.dot_general(
        node_ref[...], edge_ref[...],
        (((1,), (1,)), ((), ())),
        preferred_element_type=jnp.float32,
    )


_matmul_b = pl.pallas_call(
    _mm_b_body,
    grid=(N_HALF // M_BLK,),
    in_specs=[
        pl.BlockSpec((M_BLK, D), lambda i: (i, 0)),
        pl.BlockSpec((N_EDGE, D), lambda i: (0, 0)),
        pl.BlockSpec(memory_space=pltpu.MemorySpace.HBM),
    ],
    out_specs=pl.BlockSpec((M_BLK, N_EDGE),
                           lambda i: (i + N_HALF // M_BLK, 0)),
    out_shape=jax.ShapeDtypeStruct((N_NODE, N_EDGE), jnp.float32),
    input_output_aliases={2: 0},
)


def kernel(node_labels, hyperedge_labels, embedding, edge_embedding):
    node_tabT = embedding.T
    edge_tabT = edge_embedding.T
    nodes_a, edge_embeds = _gather_a(node_labels[:N_HALF], hyperedge_labels,
                                     node_tabT, edge_tabT)
    nodes_b = _gather_b(node_labels[N_HALF:], node_tabT)
    out1, j2d = _matmul_a(nodes_a, edge_embeds)
    recon_logits = _matmul_b(nodes_b, edge_embeds, out1)
    return recon_logits, j2d.reshape(D)
